# baseline (speedup 1.0000x reference)
import functools

import jax
import jax.numpy as jnp
from jax import lax
from jax.experimental import pallas as pl
from jax.experimental.pallas import tpu as pltpu
from jax.experimental.pallas import tpu_sc as plsc

N = 10000
D = 128
E = 320000
NC = 2
NS = 16
NW = NC * NS
CHR = 125
CHW = 128
NCHUNK = 80
NJ = NCHUNK // 2
NP = 10112
ROWS_T = NP // NS


def _deg_body(dst_hbm, ones_hbm, zeros_hbm, out_hbm,
              ia, ib, ones_v, acc_sh, sia, sib, ssa, ssb):
  c = lax.axis_index("c")
  s = lax.axis_index("s")
  wid = s * NC + c
  pltpu.sync_copy(ones_hbm, ones_v)
  pltpu.sync_copy(zeros_hbm.at[pl.ds(s * ROWS_T, ROWS_T)],
                  acc_sh.at[pl.ds(s * ROWS_T, ROWS_T)])
  plsc.subcore_barrier()
  pltpu.async_copy(dst_hbm.at[wid, 0], ia, sia)
  pltpu.async_copy(dst_hbm.at[wid, 1], ib, sib)

  def body(j, carry):
    i0 = 2 * j
    pltpu.make_async_copy(dst_hbm.at[wid, i0], ia, sia).wait()
    pltpu.async_copy(ones_v, acc_sh.at[ia.at[0]], ssa, add=True)
    pltpu.make_async_copy(dst_hbm.at[wid, i0 + 1], ib, sib).wait()
    pltpu.async_copy(ones_v, acc_sh.at[ib.at[0]], ssb, add=True)
    pltpu.make_async_copy(ones_v, acc_sh.at[ia.at[0]], ssa).wait()

    @pl.when(j + 1 < NJ)
    def _():
      pltpu.async_copy(dst_hbm.at[wid, i0 + 2], ia, sia)

    pltpu.make_async_copy(ones_v, acc_sh.at[ib.at[0]], ssb).wait()

    @pl.when(j + 1 < NJ)
    def _():
      pltpu.async_copy(dst_hbm.at[wid, i0 + 3], ib, sib)

    return carry

  lax.fori_loop(0, NJ, body, 0)
  plsc.subcore_barrier()
  pltpu.sync_copy(acc_sh.at[pl.ds(s * ROWS_T, ROWS_T)],
                  out_hbm.at[c, pl.ds(s * ROWS_T, ROWS_T)])


@functools.cache
def _deg_kernel():
  mesh = plsc.VectorSubcoreMesh(
      core_axis_name="c", subcore_axis_name="s", num_cores=NC, num_subcores=NS)
  return pl.kernel(
      _deg_body,
      out_type=jax.ShapeDtypeStruct((NC, NP, D), jnp.float32),
      mesh=mesh,
      scratch_types=[
          pltpu.VMEM((1, CHW), jnp.int32),
          pltpu.VMEM((1, CHW), jnp.int32),
          pltpu.VMEM((CHW, D), jnp.float32),
          pltpu.VMEM_SHARED((NP, D), jnp.float32),
          pltpu.SemaphoreType.DMA,
          pltpu.SemaphoreType.DMA,
          pltpu.SemaphoreType.DMA,
          pltpu.SemaphoreType.DMA,
      ],
  )


def _gs_body(src_hbm, dst_hbm, u_hbm, zeros_hbm, out_hbm,
             ias, iad, ibs, ibd, rows0, rows1, acc_sh,
             sias, siad, sibs, sibd, sg0, sg1, ss0, ss1):
  c = lax.axis_index("c")
  s = lax.axis_index("s")
  wid = s * NC + c
  pltpu.sync_copy(zeros_hbm.at[pl.ds(s * ROWS_T, ROWS_T)],
                  acc_sh.at[pl.ds(s * ROWS_T, ROWS_T)])
  plsc.subcore_barrier()
  pltpu.sync_copy(src_hbm.at[wid, 0], ias)
  pltpu.async_copy(u_hbm.at[ias.at[0]], rows0, sg0)
  pltpu.async_copy(dst_hbm.at[wid, 0], iad, siad)
  pltpu.async_copy(src_hbm.at[wid, 1], ibs, sibs)
  pltpu.async_copy(dst_hbm.at[wid, 1], ibd, sibd)

  def body(j, carry):
    i0 = 2 * j
    pltpu.make_async_copy(src_hbm.at[wid, i0 + 1], ibs, sibs).wait()
    pltpu.async_copy(u_hbm.at[ibs.at[0]], rows1, sg1)
    pltpu.make_async_copy(u_hbm.at[ias.at[0]], rows0, sg0).wait()

    @pl.when(j + 1 < NJ)
    def _():
      pltpu.async_copy(src_hbm.at[wid, i0 + 2], ias, sias)

    pltpu.make_async_copy(dst_hbm.at[wid, i0], iad, siad).wait()
    pltpu.async_copy(rows0, acc_sh.at[iad.at[0]], ss0, add=True)
    pltpu.make_async_copy(u_hbm.at[ibs.at[0]], rows1, sg1).wait()

    @pl.when(j + 1 < NJ)
    def _():
      pltpu.async_copy(src_hbm.at[wid, i0 + 3], ibs, sibs)

    pltpu.make_async_copy(dst_hbm.at[wid, i0 + 1], ibd, sibd).wait()
    pltpu.async_copy(rows1, acc_sh.at[ibd.at[0]], ss1, add=True)
    pltpu.make_async_copy(rows0, acc_sh.at[iad.at[0]], ss0).wait()

    @pl.when(j + 1 < NJ)
    def _():
      pltpu.make_async_copy(src_hbm.at[wid, i0 + 2], ias, sias).wait()
      pltpu.async_copy(u_hbm.at[ias.at[0]], rows0, sg0)
      pltpu.async_copy(dst_hbm.at[wid, i0 + 2], iad, siad)

    pltpu.make_async_copy(rows1, acc_sh.at[ibd.at[0]], ss1).wait()

    @pl.when(j + 1 < NJ)
    def _():
      pltpu.async_copy(dst_hbm.at[wid, i0 + 3], ibd, sibd)

    return carry

  lax.fori_loop(0, NJ, body, 0)
  plsc.subcore_barrier()
  pltpu.sync_copy(acc_sh.at[pl.ds(s * ROWS_T, ROWS_T)],
                  out_hbm.at[c, pl.ds(s * ROWS_T, ROWS_T)])


@functools.cache
def _gs_kernel():
  mesh = plsc.VectorSubcoreMesh(
      core_axis_name="c", subcore_axis_name="s", num_cores=NC, num_subcores=NS)
  return pl.kernel(
      _gs_body,
      out_type=jax.ShapeDtypeStruct((NC, NP, D), jnp.float32),
      mesh=mesh,
      scratch_types=[
          pltpu.VMEM((1, CHW), jnp.int32),
          pltpu.VMEM((1, CHW), jnp.int32),
          pltpu.VMEM((1, CHW), jnp.int32),
          pltpu.VMEM((1, CHW), jnp.int32),
          pltpu.VMEM((CHW, D), jnp.float32),
          pltpu.VMEM((CHW, D), jnp.float32),
          pltpu.VMEM_SHARED((NP, D), jnp.float32),
          pltpu.SemaphoreType.DMA,
          pltpu.SemaphoreType.DMA,
          pltpu.SemaphoreType.DMA,
          pltpu.SemaphoreType.DMA,
          pltpu.SemaphoreType.DMA,
          pltpu.SemaphoreType.DMA,
          pltpu.SemaphoreType.DMA,
          pltpu.SemaphoreType.DMA,
      ],
  )


def _tc_head_body(degp_ref, x_ref, w_ref, u_ref, dis_ref):
  degp = degp_ref[...]
  deg = jnp.sum(degp[0, :N] + degp[1, :N], axis=1, keepdims=True) / D + 1.0
  dis = lax.rsqrt(deg)
  u_ref[...] = dis * jnp.dot(x_ref[...], w_ref[...],
                             preferred_element_type=jnp.float32)
  dis_ref[...] = dis


_tc_head = pl.pallas_call(
    _tc_head_body,
    out_shape=(jax.ShapeDtypeStruct((N, D), jnp.float32),
               jax.ShapeDtypeStruct((N, 1), jnp.float32)),
)


def _bn_relu(p, u, dis, b, g, be):
  y = jax.nn.relu(dis * (p[0, :N] + p[1, :N] + u) + b)
  m = jnp.mean(y, axis=0, keepdims=True)
  yc = y - m
  v = jnp.mean(yc * yc, axis=0, keepdims=True)
  return g * yc * lax.rsqrt(v + 1e-5) + be


def _tc_mid_body(p_ref, u_ref, dis_ref, b_ref, g_ref, be_ref, w_ref, unext_ref):
  dis = dis_ref[...]
  xbn = _bn_relu(p_ref[...], u_ref[...], dis, b_ref[...], g_ref[...],
                 be_ref[...])
  unext_ref[...] = dis * jnp.dot(xbn, w_ref[...],
                                 preferred_element_type=jnp.float32)


_tc_mid = pl.pallas_call(
    _tc_mid_body,
    out_shape=jax.ShapeDtypeStruct((N, D), jnp.float32),
)


def _tc_tail_body(p_ref, u_ref, dis_ref, b_ref, g_ref, be_ref, out_ref):
  out_ref[...] = _bn_relu(p_ref[...], u_ref[...], dis_ref[...], b_ref[...],
                          g_ref[...], be_ref[...])


_tc_tail = pl.pallas_call(
    _tc_tail_body,
    out_shape=jax.ShapeDtypeStruct((N, D), jnp.float32),
)


def _pack_edges(ei):
  npad = CHW - CHR
  src = ei[0].reshape(NW, NCHUNK, CHR)
  dst = ei[1].reshape(NW, NCHUNK, CHR)
  base = (jnp.arange(NW * NCHUNK, dtype=jnp.int32) * 7).reshape(NW, NCHUNK, 1)
  off = jnp.arange(npad, dtype=jnp.int32).reshape(1, 1, npad)
  pad_src = (base + off) % N
  pad_dst = N + (base + off) % (NP - N)
  src = jnp.concatenate([src, pad_src], axis=2).reshape(NW, NCHUNK, 1, CHW)
  dst = jnp.concatenate([dst, pad_dst], axis=2).reshape(NW, NCHUNK, 1, CHW)
  return src, dst


@jax.jit
def kernel(edge_index, node_attr, edge_attr,
           W1, b1, g1, be1, W2, b2, g2, be2, W3, b3, g3, be3):
  del edge_attr
  src4, dst4 = _pack_edges(edge_index.astype(jnp.int32))
  zeros_d = jnp.zeros((NP, D), jnp.float32)
  ones_w = jnp.ones((CHW, D), jnp.float32)
  row = lambda a: a.reshape(1, D)

  degp = _deg_kernel()(dst4, ones_w, zeros_d)
  u1, dis = _tc_head(degp, node_attr, W1)
  gs = _gs_kernel()
  p1 = gs(src4, dst4, u1, zeros_d)
  u2 = _tc_mid(p1, u1, dis, row(b1), row(g1), row(be1), W2)
  p2 = gs(src4, dst4, u2, zeros_d)
  u3 = _tc_mid(p2, u2, dis, row(b2), row(g2), row(be2), W3)
  p3 = gs(src4, dst4, u3, zeros_d)
  return _tc_tail(p3, u3, dis, row(b3), row(g3), row(be3))


# speedup vs baseline: 1.0441x; 1.0441x over previous
import functools

import jax
import jax.numpy as jnp
from jax import lax
from jax.experimental import pallas as pl
from jax.experimental.pallas import tpu as pltpu
from jax.experimental.pallas import tpu_sc as plsc

N = 10000
D = 128
E = 320000
NC = 2
NS = 16
NW = NC * NS
CHR = 125
CHW = 128
NCHUNK = 80
NJ = NCHUNK // 2
NP = 10112
ROWS_T = NP // NS


def _deg_body(dst_hbm, ones_hbm, zeros_hbm, out_hbm,
              ia, ib, ones_v, acc_sh, sia, sib, ssa, ssb):
  c = lax.axis_index("c")
  s = lax.axis_index("s")
  wid = s * NC + c
  pltpu.sync_copy(ones_hbm, ones_v)
  pltpu.sync_copy(zeros_hbm.at[pl.ds(s * ROWS_T, ROWS_T)],
                  acc_sh.at[pl.ds(s * ROWS_T, ROWS_T)])
  plsc.subcore_barrier()
  pltpu.async_copy(dst_hbm.at[wid, 0], ia, sia)
  pltpu.async_copy(dst_hbm.at[wid, 1], ib, sib)

  def body(j, carry):
    i0 = 2 * j
    pltpu.make_async_copy(dst_hbm.at[wid, i0], ia, sia).wait()
    pltpu.async_copy(ones_v, acc_sh.at[ia.at[0]], ssa, add=True)
    pltpu.make_async_copy(dst_hbm.at[wid, i0 + 1], ib, sib).wait()
    pltpu.async_copy(ones_v, acc_sh.at[ib.at[0]], ssb, add=True)
    pltpu.make_async_copy(ones_v, acc_sh.at[ia.at[0]], ssa).wait()

    @pl.when(j + 1 < NJ)
    def _():
      pltpu.async_copy(dst_hbm.at[wid, i0 + 2], ia, sia)

    pltpu.make_async_copy(ones_v, acc_sh.at[ib.at[0]], ssb).wait()

    @pl.when(j + 1 < NJ)
    def _():
      pltpu.async_copy(dst_hbm.at[wid, i0 + 3], ib, sib)

    return carry

  lax.fori_loop(0, NJ, body, 0)
  plsc.subcore_barrier()
  pltpu.sync_copy(acc_sh.at[pl.ds(s * ROWS_T, ROWS_T)],
                  out_hbm.at[c, pl.ds(s * ROWS_T, ROWS_T)])


@functools.cache
def _deg_kernel():
  mesh = plsc.VectorSubcoreMesh(
      core_axis_name="c", subcore_axis_name="s", num_cores=NC, num_subcores=NS)
  return pl.kernel(
      _deg_body,
      out_type=jax.ShapeDtypeStruct((NC, NP, D), jnp.float32),
      mesh=mesh,
      scratch_types=[
          pltpu.VMEM((1, CHW), jnp.int32),
          pltpu.VMEM((1, CHW), jnp.int32),
          pltpu.VMEM((CHW, D), jnp.float32),
          pltpu.VMEM_SHARED((NP, D), jnp.float32),
          pltpu.SemaphoreType.DMA,
          pltpu.SemaphoreType.DMA,
          pltpu.SemaphoreType.DMA,
          pltpu.SemaphoreType.DMA,
      ],
  )


def _gs_body(idx_hbm, u_hbm, zeros_hbm, out_hbm,
             ia, ib, rows0, rows1, acc_sh, sib, sg0, sg1, ss0, ss1):
  c = lax.axis_index("c")
  s = lax.axis_index("s")
  wid = s * NC + c
  pltpu.sync_copy(zeros_hbm.at[pl.ds(s * ROWS_T, ROWS_T)],
                  acc_sh.at[pl.ds(s * ROWS_T, ROWS_T)])
  plsc.subcore_barrier()
  pltpu.sync_copy(idx_hbm.at[wid, 0], ia)
  pltpu.async_copy(u_hbm.at[ia.at[0]], rows0, sg0)
  pltpu.async_copy(idx_hbm.at[wid, 1], ib, sib)

  def body(j, carry):
    i0 = 2 * j
    pltpu.make_async_copy(idx_hbm.at[wid, i0 + 1], ib, sib).wait()
    pltpu.async_copy(u_hbm.at[ib.at[0]], rows1, sg1)
    pltpu.make_async_copy(u_hbm.at[ia.at[0]], rows0, sg0).wait()
    pltpu.async_copy(rows0, acc_sh.at[ia.at[1]], ss0, add=True)
    pltpu.make_async_copy(u_hbm.at[ib.at[0]], rows1, sg1).wait()
    pltpu.async_copy(rows1, acc_sh.at[ib.at[1]], ss1, add=True)
    pltpu.make_async_copy(rows0, acc_sh.at[ia.at[1]], ss0).wait()

    @pl.when(j + 1 < NJ)
    def _():
      pltpu.sync_copy(idx_hbm.at[wid, i0 + 2], ia)
      pltpu.async_copy(u_hbm.at[ia.at[0]], rows0, sg0)

    pltpu.make_async_copy(rows1, acc_sh.at[ib.at[1]], ss1).wait()

    @pl.when(j + 1 < NJ)
    def _():
      pltpu.async_copy(idx_hbm.at[wid, i0 + 3], ib, sib)

    return carry

  lax.fori_loop(0, NJ, body, 0)
  plsc.subcore_barrier()
  pltpu.sync_copy(acc_sh.at[pl.ds(s * ROWS_T, ROWS_T)],
                  out_hbm.at[c, pl.ds(s * ROWS_T, ROWS_T)])


@functools.cache
def _gs_kernel():
  mesh = plsc.VectorSubcoreMesh(
      core_axis_name="c", subcore_axis_name="s", num_cores=NC, num_subcores=NS)
  return pl.kernel(
      _gs_body,
      out_type=jax.ShapeDtypeStruct((NC, NP, D), jnp.float32),
      mesh=mesh,
      scratch_types=[
          pltpu.VMEM((2, CHW), jnp.int32),
          pltpu.VMEM((2, CHW), jnp.int32),
          pltpu.VMEM((CHW, D), jnp.float32),
          pltpu.VMEM((CHW, D), jnp.float32),
          pltpu.VMEM_SHARED((NP, D), jnp.float32),
          pltpu.SemaphoreType.DMA,
          pltpu.SemaphoreType.DMA,
          pltpu.SemaphoreType.DMA,
          pltpu.SemaphoreType.DMA,
          pltpu.SemaphoreType.DMA,
      ],
  )


def _tc_head_body(degp_ref, x_ref, w_ref, u_ref, dis_ref):
  degp = degp_ref[...]
  deg = jnp.sum(degp[0, :N] + degp[1, :N], axis=1, keepdims=True) / D + 1.0
  dis = lax.rsqrt(deg)
  u_ref[...] = dis * jnp.dot(x_ref[...], w_ref[...],
                             preferred_element_type=jnp.float32)
  dis_ref[...] = dis


_tc_head = pl.pallas_call(
    _tc_head_body,
    out_shape=(jax.ShapeDtypeStruct((N, D), jnp.float32),
               jax.ShapeDtypeStruct((N, 1), jnp.float32)),
)


def _bn_relu(p, u, dis, b, g, be):
  y = jax.nn.relu(dis * (p[0, :N] + p[1, :N] + u) + b)
  m = jnp.mean(y, axis=0, keepdims=True)
  yc = y - m
  v = jnp.mean(yc * yc, axis=0, keepdims=True)
  return g * yc * lax.rsqrt(v + 1e-5) + be


def _tc_mid_body(p_ref, u_ref, dis_ref, b_ref, g_ref, be_ref, w_ref, unext_ref):
  dis = dis_ref[...]
  xbn = _bn_relu(p_ref[...], u_ref[...], dis, b_ref[...], g_ref[...],
                 be_ref[...])
  unext_ref[...] = dis * jnp.dot(xbn, w_ref[...],
                                 preferred_element_type=jnp.float32)


_tc_mid = pl.pallas_call(
    _tc_mid_body,
    out_shape=jax.ShapeDtypeStruct((N, D), jnp.float32),
)


def _tc_tail_body(p_ref, u_ref, dis_ref, b_ref, g_ref, be_ref, out_ref):
  out_ref[...] = _bn_relu(p_ref[...], u_ref[...], dis_ref[...], b_ref[...],
                          g_ref[...], be_ref[...])


_tc_tail = pl.pallas_call(
    _tc_tail_body,
    out_shape=jax.ShapeDtypeStruct((N, D), jnp.float32),
)


def _pack_edges(ei):
  npad = CHW - CHR
  src = ei[0].reshape(NW, NCHUNK, CHR)
  dst = ei[1].reshape(NW, NCHUNK, CHR)
  base = (jnp.arange(NW * NCHUNK, dtype=jnp.int32) * 7).reshape(NW, NCHUNK, 1)
  off = jnp.arange(npad, dtype=jnp.int32).reshape(1, 1, npad)
  pad_src = (base + off) % N
  pad_dst = N + (base + off) % (NP - N)
  src = jnp.concatenate([src, pad_src], axis=2)
  dst = jnp.concatenate([dst, pad_dst], axis=2)
  idx4 = jnp.stack([src, dst], axis=2)
  return idx4, dst.reshape(NW, NCHUNK, 1, CHW)


@jax.jit
def kernel(edge_index, node_attr, edge_attr,
           W1, b1, g1, be1, W2, b2, g2, be2, W3, b3, g3, be3):
  del edge_attr
  idx4, dst4 = _pack_edges(edge_index.astype(jnp.int32))
  zeros_d = jnp.zeros((NP, D), jnp.float32)
  ones_w = jnp.ones((CHW, D), jnp.float32)
  row = lambda a: a.reshape(1, D)

  degp = _deg_kernel()(dst4, ones_w, zeros_d)
  u1, dis = _tc_head(degp, node_attr, W1)
  gs = _gs_kernel()
  p1 = gs(idx4, u1, zeros_d)
  u2 = _tc_mid(p1, u1, dis, row(b1), row(g1), row(be1), W2)
  p2 = gs(idx4, u2, zeros_d)
  u3 = _tc_mid(p2, u2, dis, row(b2), row(g2), row(be2), W3)
  p3 = gs(idx4, u3, zeros_d)
  return _tc_tail(p3, u3, dis, row(b3), row(g3), row(be3))


# speedup vs baseline: 1.0716x; 1.0263x over previous
import functools

import jax
import jax.numpy as jnp
from jax import lax
from jax.experimental import pallas as pl
from jax.experimental.pallas import tpu as pltpu
from jax.experimental.pallas import tpu_sc as plsc

N = 10000
D = 128
E = 320000
NC = 2
NS = 16
NW = NC * NS
CHR = 125
CHW = 128
NCHUNK = 80
NJ = NCHUNK // 2
NP = 10112
ROWS_T = NP // NS


def _deg_body(dst_hbm, ones_hbm, zeros_hbm, out_hbm,
              i0v, i1v, i2v, i3v, ones_v, acc_sh,
              si0, si1, si2, si3, ss0, ss1, ss2, ss3):
  c = lax.axis_index("c")
  s = lax.axis_index("s")
  wid = s * NC + c
  bufs = (i0v, i1v, i2v, i3v)
  sidx = (si0, si1, si2, si3)
  ssc = (ss0, ss1, ss2, ss3)
  for k in range(4):
    pltpu.async_copy(dst_hbm.at[wid, k], bufs[k], sidx[k])
  pltpu.sync_copy(ones_hbm, ones_v)
  pltpu.sync_copy(zeros_hbm.at[pl.ds(s * ROWS_T, ROWS_T)],
                  acc_sh.at[pl.ds(s * ROWS_T, ROWS_T)])
  plsc.subcore_barrier()

  def body(j, carry):
    i0 = 4 * j
    for k in range(4):
      pltpu.make_async_copy(dst_hbm.at[wid, i0 + k], bufs[k], sidx[k]).wait()
      pltpu.async_copy(ones_v, acc_sh.at[bufs[k].at[0]], ssc[k], add=True)
    for k in range(4):
      pltpu.make_async_copy(ones_v, acc_sh.at[bufs[k].at[0]], ssc[k]).wait()

      @pl.when(j + 1 < NCHUNK // 4)
      def _():
        pltpu.async_copy(dst_hbm.at[wid, i0 + 4 + k], bufs[k], sidx[k])

    return carry

  lax.fori_loop(0, NCHUNK // 4, body, 0)
  plsc.subcore_barrier()
  pltpu.sync_copy(acc_sh.at[pl.ds(s * ROWS_T, ROWS_T)],
                  out_hbm.at[c, pl.ds(s * ROWS_T, ROWS_T)])


@functools.cache
def _deg_kernel():
  mesh = plsc.VectorSubcoreMesh(
      core_axis_name="c", subcore_axis_name="s", num_cores=NC, num_subcores=NS)
  return pl.kernel(
      _deg_body,
      out_type=jax.ShapeDtypeStruct((NC, NP, D), jnp.float32),
      mesh=mesh,
      scratch_types=[
          pltpu.VMEM((1, CHW), jnp.int32),
          pltpu.VMEM((1, CHW), jnp.int32),
          pltpu.VMEM((1, CHW), jnp.int32),
          pltpu.VMEM((1, CHW), jnp.int32),
          pltpu.VMEM((CHW, D), jnp.float32),
          pltpu.VMEM_SHARED((NP, D), jnp.float32),
          pltpu.SemaphoreType.DMA,
          pltpu.SemaphoreType.DMA,
          pltpu.SemaphoreType.DMA,
          pltpu.SemaphoreType.DMA,
          pltpu.SemaphoreType.DMA,
          pltpu.SemaphoreType.DMA,
          pltpu.SemaphoreType.DMA,
          pltpu.SemaphoreType.DMA,
      ],
  )


def _gs_body(idx_hbm, u_hbm, zeros_hbm, out_hbm,
             ia, ib, rows0, rows1, acc_sh, sib, sg0, sg1, ss0, ss1):
  c = lax.axis_index("c")
  s = lax.axis_index("s")
  wid = s * NC + c
  pltpu.sync_copy(idx_hbm.at[wid, 0], ia)
  pltpu.async_copy(u_hbm.at[ia.at[0]], rows0, sg0)
  pltpu.async_copy(idx_hbm.at[wid, 1], ib, sib)
  pltpu.sync_copy(zeros_hbm.at[pl.ds(s * ROWS_T, ROWS_T)],
                  acc_sh.at[pl.ds(s * ROWS_T, ROWS_T)])
  plsc.subcore_barrier()

  def body(j, carry):
    i0 = 2 * j
    pltpu.make_async_copy(idx_hbm.at[wid, i0 + 1], ib, sib).wait()
    pltpu.async_copy(u_hbm.at[ib.at[0]], rows1, sg1)
    pltpu.make_async_copy(u_hbm.at[ia.at[0]], rows0, sg0).wait()
    pltpu.async_copy(rows0, acc_sh.at[ia.at[1]], ss0, add=True)
    pltpu.make_async_copy(u_hbm.at[ib.at[0]], rows1, sg1).wait()
    pltpu.async_copy(rows1, acc_sh.at[ib.at[1]], ss1, add=True)
    pltpu.make_async_copy(rows0, acc_sh.at[ia.at[1]], ss0).wait()

    @pl.when(j + 1 < NJ)
    def _():
      pltpu.sync_copy(idx_hbm.at[wid, i0 + 2], ia)
      pltpu.async_copy(u_hbm.at[ia.at[0]], rows0, sg0)

    pltpu.make_async_copy(rows1, acc_sh.at[ib.at[1]], ss1).wait()

    @pl.when(j + 1 < NJ)
    def _():
      pltpu.async_copy(idx_hbm.at[wid, i0 + 3], ib, sib)

    return carry

  lax.fori_loop(0, NJ, body, 0)
  plsc.subcore_barrier()
  pltpu.sync_copy(acc_sh.at[pl.ds(s * ROWS_T, ROWS_T)],
                  out_hbm.at[c, pl.ds(s * ROWS_T, ROWS_T)])


@functools.cache
def _gs_kernel():
  mesh = plsc.VectorSubcoreMesh(
      core_axis_name="c", subcore_axis_name="s", num_cores=NC, num_subcores=NS)
  return pl.kernel(
      _gs_body,
      out_type=jax.ShapeDtypeStruct((NC, NP, D), jnp.float32),
      mesh=mesh,
      scratch_types=[
          pltpu.VMEM((2, CHW), jnp.int32),
          pltpu.VMEM((2, CHW), jnp.int32),
          pltpu.VMEM((CHW, D), jnp.float32),
          pltpu.VMEM((CHW, D), jnp.float32),
          pltpu.VMEM_SHARED((NP, D), jnp.float32),
          pltpu.SemaphoreType.DMA,
          pltpu.SemaphoreType.DMA,
          pltpu.SemaphoreType.DMA,
          pltpu.SemaphoreType.DMA,
          pltpu.SemaphoreType.DMA,
      ],
  )


def _tc_head_body(degp_ref, x_ref, w_ref, u_ref, dis_ref):
  degp = degp_ref[...]
  deg = jnp.sum(degp[0, :N] + degp[1, :N], axis=1, keepdims=True) / D + 1.0
  dis = lax.rsqrt(deg)
  u_ref[...] = dis * jnp.dot(x_ref[...], w_ref[...],
                             preferred_element_type=jnp.float32)
  dis_ref[...] = dis


_tc_head = pl.pallas_call(
    _tc_head_body,
    out_shape=(jax.ShapeDtypeStruct((N, D), jnp.float32),
               jax.ShapeDtypeStruct((N, 1), jnp.float32)),
)


def _bn_relu(p, u, dis, b, g, be):
  y = jax.nn.relu(dis * (p[0, :N] + p[1, :N] + u) + b)
  m = jnp.mean(y, axis=0, keepdims=True)
  yc = y - m
  v = jnp.mean(yc * yc, axis=0, keepdims=True)
  return g * yc * lax.rsqrt(v + 1e-5) + be


def _tc_mid_body(p_ref, u_ref, dis_ref, b_ref, g_ref, be_ref, w_ref, unext_ref):
  dis = dis_ref[...]
  xbn = _bn_relu(p_ref[...], u_ref[...], dis, b_ref[...], g_ref[...],
                 be_ref[...])
  unext_ref[...] = dis * jnp.dot(xbn, w_ref[...],
                                 preferred_element_type=jnp.float32)


_tc_mid = pl.pallas_call(
    _tc_mid_body,
    out_shape=jax.ShapeDtypeStruct((N, D), jnp.float32),
)


def _tc_tail_body(p_ref, u_ref, dis_ref, b_ref, g_ref, be_ref, out_ref):
  out_ref[...] = _bn_relu(p_ref[...], u_ref[...], dis_ref[...], b_ref[...],
                          g_ref[...], be_ref[...])


_tc_tail = pl.pallas_call(
    _tc_tail_body,
    out_shape=jax.ShapeDtypeStruct((N, D), jnp.float32),
)


def _pack_edges(ei):
  npad = CHW - CHR
  src = ei[0].reshape(NW, NCHUNK, CHR)
  dst = ei[1].reshape(NW, NCHUNK, CHR)
  base = (jnp.arange(NW * NCHUNK, dtype=jnp.int32) * 7).reshape(NW, NCHUNK, 1)
  off = jnp.arange(npad, dtype=jnp.int32).reshape(1, 1, npad)
  pad_src = (base + off) % N
  pad_dst = N + (base + off) % (NP - N)
  src = jnp.concatenate([src, pad_src], axis=2)
  dst = jnp.concatenate([dst, pad_dst], axis=2)
  idx4 = jnp.stack([src, dst], axis=2)
  return idx4, dst.reshape(NW, NCHUNK, 1, CHW)


@jax.jit
def kernel(edge_index, node_attr, edge_attr,
           W1, b1, g1, be1, W2, b2, g2, be2, W3, b3, g3, be3):
  del edge_attr
  idx4, dst4 = _pack_edges(edge_index.astype(jnp.int32))
  zeros_d = jnp.zeros((NP, D), jnp.float32)
  ones_w = jnp.ones((CHW, D), jnp.float32)
  row = lambda a: a.reshape(1, D)

  degp = _deg_kernel()(dst4, ones_w, zeros_d)
  u1, dis = _tc_head(degp, node_attr, W1)
  gs = _gs_kernel()
  p1 = gs(idx4, u1, zeros_d)
  u2 = _tc_mid(p1, u1, dis, row(b1), row(g1), row(be1), W2)
  p2 = gs(idx4, u2, zeros_d)
  u3 = _tc_mid(p2, u2, dis, row(b2), row(g2), row(be2), W3)
  p3 = gs(idx4, u3, zeros_d)
  return _tc_tail(p3, u3, dis, row(b3), row(g3), row(be3))
